# baseline (speedup 1.0000x reference)
import functools

import jax
import jax.numpy as jnp
from jax import lax
from jax.experimental import pallas as pl
from jax.experimental.pallas import tpu as pltpu
from jax.experimental.pallas import tpu_sc as plsc

_HALVES = 4
_LANES = 16


@functools.lru_cache(maxsize=None)
def _build(B, F, V, D):
    info = plsc.get_sparse_core_info()
    NC, NS = info.num_cores, info.num_subcores
    NW = NC * NS
    total = B * F
    assert total % NW == 0
    per_w = total // NW
    assert per_w % (_HALVES * _LANES) == 0 and V % 32 == 0

    mesh = plsc.VectorSubcoreMesh(core_axis_name="c", subcore_axis_name="s")

    @functools.partial(
        pl.kernel,
        mesh=mesh,
        compiler_params=pltpu.CompilerParams(
            needs_layout_passes=False, use_tc_tiling_on_sc=False
        ),
        out_type=jax.ShapeDtypeStruct((total,), jnp.float32),
        scratch_types=[
            pltpu.VMEM((per_w,), jnp.float32),
            pltpu.VMEM((per_w,), jnp.int32),
            pltpu.VMEM((per_w,), jnp.int32),
            pltpu.VMEM((per_w,), jnp.float32),
            *([pltpu.VMEM((_LANES, 32, D), jnp.float32)] * _HALVES),
            pltpu.SemaphoreType.DMA,
        ],
    )
    def k(tab_hbm, in_hbm, out_hbm, in_v, row_v, sub_v, got_v, *ring_sem):
        ring = ring_sem[:_HALVES]
        sem = ring_sem[_HALVES]
        wid = lax.axis_index("s") * NC + lax.axis_index("c")
        base = wid * per_w
        pltpu.sync_copy(in_hbm.at[pl.ds(base, per_w)], in_v)

        lane = lax.iota(jnp.int32, _LANES)

        def index(i, _):
            off = i * _LANES
            ids = in_v[pl.ds(off, _LANES)].astype(jnp.int32)
            p = base + off + lane
            f = lax.rem(p, F)
            r = f * V + ids
            row_v[pl.ds(off, _LANES)] = lax.shift_right_logical(r, 5)
            sub_v[pl.ds(off, _LANES)] = lax.bitwise_and(r, 31)
            return 0

        lax.fori_loop(0, per_w // _LANES, index, 0)

        def dma(tile, h, l):
            return pltpu.make_async_copy(
                tab_hbm.at[tile], ring[h].at[l], sem
            )

        for h in range(_HALVES):
            rows0 = row_v[pl.ds(h * _LANES, _LANES)]
            for l in range(_LANES):
                dma(rows0[l], h, l).start()

        zero = lane * 0
        step = _HALVES * _LANES

        @pl.loop(0, per_w, step=step)
        def body(g):
            for h in range(_HALVES):
                off = g + h * _LANES
                rows = row_v[pl.ds(off, _LANES)]
                subs = sub_v[pl.ds(off, _LANES)]
                for l in range(_LANES):
                    dma(rows[l], h, l).wait()
                got_v[pl.ds(off, _LANES)] = plsc.load_gather(
                    ring[h], [lane, subs, zero]
                )

                @pl.when(g + step < per_w)
                def _():
                    nrows = row_v[pl.ds(off + step, _LANES)]
                    for l in range(_LANES):
                        dma(nrows[l], h, l).start()

        pltpu.sync_copy(got_v, out_hbm.at[pl.ds(base, per_w)])

    return k


def kernel(inputs, tables):
    B, F = inputs.shape
    Ft, V, D = tables.shape
    out_flat = _build(B, F, V, D)(
        tables.reshape(Ft * V // 32, 32, D), inputs.reshape(-1)
    )
    return out_flat.reshape(B, F)


# speedup vs baseline: 1.4402x; 1.4402x over previous
import functools

import jax
import jax.numpy as jnp
from jax import lax
from jax.experimental import pallas as pl
from jax.experimental.pallas import tpu as pltpu
from jax.experimental.pallas import tpu_sc as plsc

_HALVES = 4
_LANES = 16


@functools.lru_cache(maxsize=None)
def _build(B, F, V, D):
    info = plsc.get_sparse_core_info()
    NC, NS = info.num_cores, info.num_subcores
    NW = NC * NS
    total = B * F
    assert total % NW == 0
    per_w = total // NW
    assert per_w % (_HALVES * _LANES) == 0 and V % 8 == 0

    mesh = plsc.VectorSubcoreMesh(core_axis_name="c", subcore_axis_name="s")

    @functools.partial(
        pl.kernel,
        mesh=mesh,
        compiler_params=pltpu.CompilerParams(needs_layout_passes=False),
        out_type=jax.ShapeDtypeStruct((total,), jnp.float32),
        scratch_types=[
            pltpu.VMEM((per_w,), jnp.float32),
            pltpu.VMEM((per_w,), jnp.int32),
            pltpu.VMEM((per_w,), jnp.int32),
            pltpu.VMEM((per_w,), jnp.int32),
            pltpu.VMEM((per_w,), jnp.float32),
            *([pltpu.VMEM((_LANES, 8, D), jnp.float32)] * _HALVES),
            pltpu.SemaphoreType.DMA,
        ],
    )
    def k(tab_hbm, in_hbm, out_hbm, in_v, fld_v, row_v, sub_v, got_v, *ring_sem):
        ring = ring_sem[:_HALVES]
        sem = ring_sem[_HALVES]
        wid = lax.axis_index("s") * NC + lax.axis_index("c")
        base = wid * per_w
        pltpu.sync_copy(in_hbm.at[pl.ds(base, per_w)], in_v)

        lane = lax.iota(jnp.int32, _LANES)

        def index(i, _):
            off = i * _LANES
            ids = in_v[pl.ds(off, _LANES)].astype(jnp.int32)
            p = base + off + lane
            f = lax.rem(p, F)
            fld_v[pl.ds(off, _LANES)] = f
            row_v[pl.ds(off, _LANES)] = ids - lax.bitwise_and(ids, 7)
            sub_v[pl.ds(off, _LANES)] = lax.bitwise_and(ids, 7)
            return 0

        lax.fori_loop(0, per_w // _LANES, index, 0)

        def dma(f, v0, h, l):
            v0 = pl.multiple_of(v0, 8)
            return pltpu.make_async_copy(
                tab_hbm.at[f, pl.ds(v0, 8)], ring[h].at[l], sem
            )

        for h in range(_HALVES):
            flds0 = fld_v[pl.ds(h * _LANES, _LANES)]
            rows0 = row_v[pl.ds(h * _LANES, _LANES)]
            for l in range(_LANES):
                dma(flds0[l], rows0[l], h, l).start()

        zero = lane * 0
        step = _HALVES * _LANES

        @pl.loop(0, per_w, step=step)
        def body(g):
            for h in range(_HALVES):
                off = g + h * _LANES
                flds = fld_v[pl.ds(off, _LANES)]
                rows = row_v[pl.ds(off, _LANES)]
                subs = sub_v[pl.ds(off, _LANES)]
                for l in range(_LANES):
                    dma(flds[l], rows[l], h, l).wait()
                got_v[pl.ds(off, _LANES)] = plsc.load_gather(
                    ring[h], [lane, subs, zero]
                )

                @pl.when(g + step < per_w)
                def _():
                    nflds = fld_v[pl.ds(off + step, _LANES)]
                    nrows = row_v[pl.ds(off + step, _LANES)]
                    for l in range(_LANES):
                        dma(nflds[l], nrows[l], h, l).start()

        pltpu.sync_copy(got_v, out_hbm.at[pl.ds(base, per_w)])

    return k


def kernel(inputs, tables):
    B, F = inputs.shape
    Ft, V, D = tables.shape
    out_flat = _build(B, F, V, D)(tables, inputs.reshape(-1))
    return out_flat.reshape(B, F)


# speedup vs baseline: 2.8596x; 1.9855x over previous
import functools

import jax
import jax.numpy as jnp
from jax import lax
from jax.experimental import pallas as pl
from jax.experimental.pallas import tpu as pltpu
from jax.experimental.pallas import tpu_sc as plsc

_HALVES = 4
_LANES = 16


@functools.lru_cache(maxsize=None)
def _build(B, F, V, D):
    info = plsc.get_sparse_core_info()
    NC, NS = info.num_cores, info.num_subcores
    NW = NC * NS
    total = B * F
    assert total % NW == 0
    per_w = total // NW
    assert per_w % (_HALVES * _LANES) == 0 and V % 8 == 0

    mesh = plsc.VectorSubcoreMesh(core_axis_name="c", subcore_axis_name="s")

    @functools.partial(
        pl.kernel,
        mesh=mesh,
        compiler_params=pltpu.CompilerParams(needs_layout_passes=False),
        out_type=jax.ShapeDtypeStruct((total,), jnp.float32),
        scratch_types=[
            pltpu.VMEM((per_w,), jnp.float32),
            pltpu.VMEM((per_w,), jnp.int32),
            pltpu.VMEM((per_w,), jnp.int32),
            pltpu.VMEM((per_w,), jnp.float32),
            *([pltpu.VMEM((_LANES, 8, D), jnp.float32)] * _HALVES),
            pltpu.SemaphoreType.DMA,
        ],
    )
    def k(tab_hbm, in_hbm, out_hbm, in_v, row_v, sub_v, got_v, *ring_sem):
        ring = ring_sem[:_HALVES]
        sem = ring_sem[_HALVES]
        wid = lax.axis_index("s") * NC + lax.axis_index("c")
        base = wid * per_w
        pltpu.sync_copy(in_hbm.at[pl.ds(base, per_w)], in_v)

        lane = lax.iota(jnp.int32, _LANES)

        def index(i, _):
            off = i * _LANES
            ids = in_v[pl.ds(off, _LANES)].astype(jnp.int32)
            p = base + off + lane
            f = lax.rem(p, F)
            r = f * V + ids
            row_v[pl.ds(off, _LANES)] = lax.shift_right_logical(r, 3)
            sub_v[pl.ds(off, _LANES)] = lax.bitwise_and(r, 7)
            return 0

        lax.fori_loop(0, per_w // _LANES, index, 0)

        def dma(tile, h, l):
            return pltpu.make_async_copy(
                tab_hbm.at[tile], ring[h].at[l], sem
            )

        for h in range(_HALVES):
            rows0 = row_v[pl.ds(h * _LANES, _LANES)]
            for l in range(_LANES):
                dma(rows0[l], h, l).start()

        zero = lane * 0
        step = _HALVES * _LANES

        @pl.loop(0, per_w, step=step)
        def body(g):
            for h in range(_HALVES):
                off = g + h * _LANES
                rows = row_v[pl.ds(off, _LANES)]
                subs = sub_v[pl.ds(off, _LANES)]
                for l in range(_LANES):
                    dma(rows[l], h, l).wait()
                got_v[pl.ds(off, _LANES)] = plsc.load_gather(
                    ring[h], [lane, subs, zero]
                )

                @pl.when(g + step < per_w)
                def _():
                    nrows = row_v[pl.ds(off + step, _LANES)]
                    for l in range(_LANES):
                        dma(nrows[l], h, l).start()

        pltpu.sync_copy(got_v, out_hbm.at[pl.ds(base, per_w)])

    return k


def kernel(inputs, tables):
    B, F = inputs.shape
    Ft, V, D = tables.shape
    out_flat = _build(B, F, V, D)(
        tables.reshape(Ft * V // 8, 8, D), inputs.reshape(-1)
    )
    return out_flat.reshape(B, F)


# speedup vs baseline: 9.1242x; 3.1907x over previous
import functools

import jax
import jax.numpy as jnp
from jax import lax
from jax.experimental import pallas as pl
from jax.experimental.pallas import tpu as pltpu
from jax.experimental.pallas import tpu_sc as plsc

_CHUNK = 128
_LANES = 16


@functools.lru_cache(maxsize=None)
def _build_gather(B, F, V):
    total = B * F
    info = plsc.get_sparse_core_info()
    NC, NS = info.num_cores, info.num_subcores
    NW = NC * NS
    assert total % NW == 0
    per_w = total // NW
    assert per_w % _CHUNK == 0
    n_chunks = per_w // _CHUNK
    vecs_per_chunk = _CHUNK // _LANES

    mesh = plsc.VectorSubcoreMesh(core_axis_name="c", subcore_axis_name="s")

    @functools.partial(
        pl.kernel,
        mesh=mesh,
        out_type=jax.ShapeDtypeStruct((total,), jnp.float32),
        scratch_types=[
            pltpu.VMEM((per_w,), jnp.float32),
            pltpu.VMEM((n_chunks, _CHUNK), jnp.int32),
            pltpu.VMEM((per_w,), jnp.float32),
            pltpu.SemaphoreType.DMA,
        ],
    )
    def k(col_hbm, in_hbm, out_hbm, in_v, idx_v, got_v, sem):
        wid = lax.axis_index("s") * NC + lax.axis_index("c")
        base = wid * per_w
        pltpu.sync_copy(in_hbm.at[pl.ds(base, per_w)], in_v)

        lane = lax.iota(jnp.int32, _LANES)
        copies = []
        for j in range(n_chunks):
            def step(i, _):
                off = j * _CHUNK + i * _LANES
                ids = in_v[pl.ds(off, _LANES)].astype(jnp.int32)
                p = base + off + lane
                f = lax.rem(p, F)
                idx_v[j, pl.ds(i * _LANES, _LANES)] = f * V + ids
                return 0

            lax.fori_loop(0, vecs_per_chunk, step, 0)
            copies.append(
                pltpu.async_copy(
                    col_hbm.at[idx_v.at[j]],
                    got_v.at[pl.ds(j * _CHUNK, _CHUNK)],
                    sem,
                )
            )
        for c in copies:
            c.wait()
        pltpu.sync_copy(got_v, out_hbm.at[pl.ds(base, per_w)])

    return k


def kernel(inputs, tables):
    B, F = inputs.shape
    Ft, V, D = tables.shape
    col0 = tables[:, :, 0].reshape(-1)
    out_flat = _build_gather(B, F, V)(col0, inputs.reshape(-1))
    return out_flat.reshape(B, F)
